# baseline (speedup 1.0000x reference)
import functools

import jax
import jax.numpy as jnp
import numpy as np
from jax import lax
from jax.experimental import pallas as pl
from jax.experimental.pallas import tpu as pltpu
from jax.experimental.pallas import tpu_sc as plsc

_FIELD_DIMS = [100000] * 26
_OFF = np.concatenate([[0], np.cumsum(_FIELD_DIMS)[:-1]]).astype(np.int32)
_B, _F, _E = 16384, 26, 16
_V = int(sum(_FIELD_DIMS))
_NC, _NS = 2, 16
_NW = _NC * _NS
_BPW = _B // _NW
_CB = 128
_NCHUNK = _BPW // _CB

_W = 1024
_VMAIN = (_V // _W) * _W
_NDCHUNK = _VMAIN // _W
_DPW = -(-_NDCHUNK // _NW)
_NBUF = 3
_DTRIP = -(-_DPW // _NBUF)
_TAIL = _V - _VMAIN


@functools.partial(
    pl.kernel,
    out_type=jax.ShapeDtypeStruct((_V * _E,), jnp.float32),
    mesh=plsc.VectorSubcoreMesh(core_axis_name="c", subcore_axis_name="s"),
    compiler_params=pltpu.CompilerParams(
        use_tc_tiling_on_sc=True, needs_layout_passes=False),
    scratch_types=(
        [pltpu.VMEM((_E, _W), jnp.float32)] * _NBUF
        + [pltpu.VMEM((_W * _E,), jnp.float32)] * _NBUF
        + [pltpu.SemaphoreType.DMA] * (2 * _NBUF)
    ),
)
def _detile_sc(emb_t, tail_lin, out_hbm, *bufs):
    wid = lax.axis_index("s") * _NC + lax.axis_index("c")
    chunks = bufs[:_NBUF]
    rows = bufs[_NBUF:2 * _NBUF]
    sems_i = bufs[2 * _NBUF:3 * _NBUF]
    sems_o = bufs[3 * _NBUF:4 * _NBUF]

    def cid_of(k):
        return k * _NW + wid

    def in_copy(k, b):
        return pltpu.make_async_copy(
            emb_t.at[:, pl.ds(cid_of(k) * _W, _W)], chunks[b], sems_i[b])

    def out_copy(k, b):
        return pltpu.make_async_copy(
            rows[b], out_hbm.at[pl.ds(cid_of(k) * _W * _E, _W * _E)],
            sems_o[b])

    def transpose(b):
        def tbody(g, carry):
            j16 = g * 16
            pos = (j16 + lax.iota(jnp.int32, 16)) * _E
            for e in range(_E):
                vals = chunks[b][e, pl.ds(j16, 16)]
                plsc.store_scatter(rows[b], [pos + e], vals)
            return carry

        lax.fori_loop(0, _W // 16, tbody, 0)

    for b in range(_NBUF):
        @pl.when(cid_of(b) < _NDCHUNK)
        def _():
            in_copy(b, b).start()

    def ring_body(m, carry):
        for b in range(_NBUF):
            k = _NBUF * m + b

            @pl.when(cid_of(k) < _NDCHUNK)
            def _():
                in_copy(k, b).wait()

                @pl.when(m > 0)
                def _():
                    out_copy(k - _NBUF, b).wait()

                transpose(b)
                out_copy(k, b).start()

                @pl.when(cid_of(k + _NBUF) < _NDCHUNK)
                def _():
                    in_copy(k + _NBUF, b).start()

        return carry

    lax.fori_loop(0, _DTRIP, ring_body, 0)

    for k in range(max(0, _NBUF * _DTRIP - 2 * _NBUF), _NBUF * _DTRIP):
        @pl.when((cid_of(k) < _NDCHUNK) & (cid_of(k + _NBUF) >= _NDCHUNK))
        def _():
            out_copy(k, k % _NBUF).wait()

    @pl.when(wid == _NW - 1)
    def _():
        pltpu.sync_copy(tail_lin, out_hbm.at[pl.ds(_VMAIN * _E, _TAIL * _E)])


@functools.partial(
    pl.kernel,
    out_type=(
        jax.ShapeDtypeStruct((_B, _E), jnp.float32),
        jax.ShapeDtypeStruct((_B,), jnp.float32),
    ),
    mesh=plsc.VectorSubcoreMesh(core_axis_name="c", subcore_axis_name="s"),
    compiler_params=pltpu.CompilerParams(use_tc_tiling_on_sc=False),
    scratch_types=[
        pltpu.VMEM((_CB * _F,), jnp.int32),
        pltpu.VMEM((_CB * _F, _E), jnp.float32),
        pltpu.VMEM((_CB * _F,), jnp.float32),
        pltpu.VMEM((_CB, _E), jnp.float32),
        pltpu.VMEM((_CB,), jnp.float32),
        pltpu.SemaphoreType.DMA,
        pltpu.SemaphoreType.DMA,
    ],
)
def _fm_sc(idx_hbm, emb_hbm, fc_hbm, cross_hbm, lin_hbm,
           idx_v, rows_v, fc_v, cross_v, lin_v, sem_e, sem_f):
    wid = lax.axis_index("s") * _NC + lax.axis_index("c")
    for c in range(_NCHUNK):
        base = wid * _BPW + c * _CB
        pltpu.sync_copy(
            idx_hbm.at[pl.ds((wid * _NCHUNK + c) * _CB * _F, _CB * _F)], idx_v)
        cp_e = pltpu.async_copy(emb_hbm.at[idx_v], rows_v, sem_e)
        cp_f = pltpu.async_copy(fc_hbm.at[idx_v], fc_v, sem_f)
        cp_e.wait()
        cp_f.wait()

        def fm_body(b, carry):
            r = rows_v[b, :]
            s = r
            sq = r * r
            for f in range(1, _F):
                r = rows_v[f * _CB + b, :]
                s = s + r
                sq = sq + r * r
            cross_v[b, :] = 0.5 * (s * s - sq)
            return carry

        lax.fori_loop(0, _CB, fm_body, 0, unroll=2)

        def lin_body(g, carry):
            b0 = g * 16
            acc = fc_v[pl.ds(b0, 16)]
            for f in range(1, _F):
                acc = acc + fc_v[pl.ds(f * _CB + b0, 16)]
            lin_v[pl.ds(b0, 16)] = acc
            return carry

        lax.fori_loop(0, _CB // 16, lin_body, 0)

        pltpu.sync_copy(cross_v, cross_hbm.at[pl.ds(base, _CB)])
        pltpu.sync_copy(lin_v, lin_hbm.at[pl.ds(base, _CB)])


_BLK = 2048


def _mlp_tc(cross_ref, lin_ref, w1_ref, b1_ref, w2_ref, b2_ref, w3_ref,
            c_ref, out_ref):
    x = cross_ref[...]
    h = jnp.dot(x, w1_ref[...], preferred_element_type=jnp.float32)
    h = jnp.maximum(h + b1_ref[...][None, :], 0.0)
    h = jnp.dot(h, w2_ref[...], preferred_element_type=jnp.float32)
    h = jnp.maximum(h + b2_ref[...][None, :], 0.0)
    o = jnp.dot(h, w3_ref[...], preferred_element_type=jnp.float32)[:, 0]
    out_ref[...] = jax.nn.sigmoid(o + lin_ref[...] + c_ref[0])


_mlp_call = pl.pallas_call(
    _mlp_tc,
    grid=(_B // _BLK,),
    in_specs=[
        pl.BlockSpec((_BLK, _E), lambda i: (i, 0)),
        pl.BlockSpec((_BLK,), lambda i: (i,)),
        pl.BlockSpec((_E, 128), lambda i: (0, 0)),
        pl.BlockSpec((128,), lambda i: (0,)),
        pl.BlockSpec((128, 64), lambda i: (0, 0)),
        pl.BlockSpec((64,), lambda i: (0,)),
        pl.BlockSpec((64, 1), lambda i: (0, 0)),
        pl.BlockSpec(memory_space=pltpu.SMEM),
    ],
    out_specs=pl.BlockSpec((_BLK,), lambda i: (i,)),
    out_shape=jax.ShapeDtypeStruct((_B,), jnp.float32),
)


def kernel(data, embed_table, fc_table, fc_bias, W1, b1, W2, b2, W3, b3):
    idx = data.astype(jnp.int32) + jnp.asarray(_OFF)[None, :]
    idx_t = idx.reshape(_NW, _NCHUNK, _CB, _F).transpose(0, 1, 3, 2).reshape(-1)
    tail_lin = lax.slice(embed_table, (_VMAIN, 0), (_V, _E)).reshape(-1)
    emb_lin = _detile_sc(embed_table.T, tail_lin).reshape(_V, _E)
    cross, lin = _fm_sc(idx_t, emb_lin, fc_table.reshape(-1))
    c = (fc_bias + b3).astype(jnp.float32)
    return _mlp_call(cross, lin, W1, b1, W2, b2, W3, c)


# speedup vs baseline: 1.0073x; 1.0073x over previous
import functools

import jax
import jax.numpy as jnp
import numpy as np
from jax import lax
from jax.experimental import pallas as pl
from jax.experimental.pallas import tpu as pltpu
from jax.experimental.pallas import tpu_sc as plsc

_FIELD_DIMS = [100000] * 26
_OFF = np.concatenate([[0], np.cumsum(_FIELD_DIMS)[:-1]]).astype(np.int32)
_B, _F, _E = 16384, 26, 16
_V = int(sum(_FIELD_DIMS))
_NC, _NS = 2, 16
_NW = _NC * _NS
_BPW = _B // _NW
_CB = 64
_NCHUNK = _BPW // _CB

_W = 1024
_VMAIN = (_V // _W) * _W
_NDCHUNK = _VMAIN // _W
_DPW = -(-_NDCHUNK // _NW)
_NBUF = 3
_DTRIP = -(-_DPW // _NBUF)
_TAIL = _V - _VMAIN


@functools.partial(
    pl.kernel,
    out_type=jax.ShapeDtypeStruct((_V * _E,), jnp.float32),
    mesh=plsc.VectorSubcoreMesh(core_axis_name="c", subcore_axis_name="s"),
    compiler_params=pltpu.CompilerParams(
        use_tc_tiling_on_sc=True, needs_layout_passes=False),
    scratch_types=(
        [pltpu.VMEM((_E, _W), jnp.float32)] * _NBUF
        + [pltpu.VMEM((_W * _E,), jnp.float32)] * _NBUF
        + [pltpu.SemaphoreType.DMA] * (2 * _NBUF)
    ),
)
def _detile_sc(emb_t, tail_lin, out_hbm, *bufs):
    wid = lax.axis_index("s") * _NC + lax.axis_index("c")
    chunks = bufs[:_NBUF]
    rows = bufs[_NBUF:2 * _NBUF]
    sems_i = bufs[2 * _NBUF:3 * _NBUF]
    sems_o = bufs[3 * _NBUF:4 * _NBUF]

    def cid_of(k):
        return k * _NW + wid

    def in_copy(k, b):
        return pltpu.make_async_copy(
            emb_t.at[:, pl.ds(cid_of(k) * _W, _W)], chunks[b], sems_i[b])

    def out_copy(k, b):
        return pltpu.make_async_copy(
            rows[b], out_hbm.at[pl.ds(cid_of(k) * _W * _E, _W * _E)],
            sems_o[b])

    def transpose(b):
        def tbody(g, carry):
            j16 = g * 16
            pos = (j16 + lax.iota(jnp.int32, 16)) * _E
            for e in range(_E):
                vals = chunks[b][e, pl.ds(j16, 16)]
                plsc.store_scatter(rows[b], [pos + e], vals)
            return carry

        lax.fori_loop(0, _W // 16, tbody, 0)

    for b in range(_NBUF):
        @pl.when(cid_of(b) < _NDCHUNK)
        def _():
            in_copy(b, b).start()

    def ring_body(m, carry):
        for b in range(_NBUF):
            k = _NBUF * m + b

            @pl.when(cid_of(k) < _NDCHUNK)
            def _():
                in_copy(k, b).wait()

                @pl.when(m > 0)
                def _():
                    out_copy(k - _NBUF, b).wait()

                transpose(b)
                out_copy(k, b).start()

                @pl.when(cid_of(k + _NBUF) < _NDCHUNK)
                def _():
                    in_copy(k + _NBUF, b).start()

        return carry

    lax.fori_loop(0, _DTRIP, ring_body, 0)

    for k in range(max(0, _NBUF * _DTRIP - 2 * _NBUF), _NBUF * _DTRIP):
        @pl.when((cid_of(k) < _NDCHUNK) & (cid_of(k + _NBUF) >= _NDCHUNK))
        def _():
            out_copy(k, k % _NBUF).wait()

    @pl.when(wid == _NW - 1)
    def _():
        pltpu.sync_copy(tail_lin, out_hbm.at[pl.ds(_VMAIN * _E, _TAIL * _E)])


@functools.partial(
    pl.kernel,
    out_type=(
        jax.ShapeDtypeStruct((_B, _E), jnp.float32),
        jax.ShapeDtypeStruct((_B,), jnp.float32),
    ),
    mesh=plsc.VectorSubcoreMesh(core_axis_name="c", subcore_axis_name="s"),
    compiler_params=pltpu.CompilerParams(use_tc_tiling_on_sc=False),
    scratch_types=(
        [pltpu.VMEM((_CB * _F,), jnp.int32)] * 2
        + [pltpu.VMEM((_CB * _F, _E), jnp.float32)] * 2
        + [pltpu.VMEM((_CB * _F,), jnp.float32)] * 2
        + [
            pltpu.VMEM((_CB, _E), jnp.float32),
            pltpu.VMEM((_CB,), jnp.float32),
        ]
        + [pltpu.SemaphoreType.DMA] * 4
    ),
)
def _fm_sc(idx_hbm, emb_hbm, fc_hbm, cross_hbm, lin_hbm,
           idx_v0, idx_v1, rows_v0, rows_v1, fc_v0, fc_v1, cross_v, lin_v,
           sem_e0, sem_e1, sem_f0, sem_f1):
    wid = lax.axis_index("s") * _NC + lax.axis_index("c")
    idxs = (idx_v0, idx_v1)
    rows = (rows_v0, rows_v1)
    fcs = (fc_v0, fc_v1)
    sems_e = (sem_e0, sem_e1)
    sems_f = (sem_f0, sem_f1)

    def stage(c, b):
        pltpu.sync_copy(
            idx_hbm.at[pl.ds((wid * _NCHUNK + c) * _CB * _F, _CB * _F)],
            idxs[b])
        pltpu.async_copy(emb_hbm.at[idxs[b]], rows[b], sems_e[b])
        pltpu.async_copy(fc_hbm.at[idxs[b]], fcs[b], sems_f[b])

    stage(0, 0)
    for c in range(_NCHUNK):
        b = c % 2
        base = wid * _BPW + c * _CB
        pltpu.make_async_copy(emb_hbm.at[idxs[b]], rows[b], sems_e[b]).wait()
        pltpu.make_async_copy(fc_hbm.at[idxs[b]], fcs[b], sems_f[b]).wait()
        if c + 1 < _NCHUNK:
            stage(c + 1, 1 - b)

        rows_v = rows[b]
        fc_v = fcs[b]

        def fm_body(bb, carry):
            r = rows_v[bb, :]
            s = r
            sq = r * r
            for f in range(1, _F):
                r = rows_v[f * _CB + bb, :]
                s = s + r
                sq = sq + r * r
            cross_v[bb, :] = 0.5 * (s * s - sq)
            return carry

        lax.fori_loop(0, _CB, fm_body, 0, unroll=2)

        def lin_body(g, carry):
            b0 = g * 16
            acc = fc_v[pl.ds(b0, 16)]
            for f in range(1, _F):
                acc = acc + fc_v[pl.ds(f * _CB + b0, 16)]
            lin_v[pl.ds(b0, 16)] = acc
            return carry

        lax.fori_loop(0, _CB // 16, lin_body, 0)

        pltpu.sync_copy(cross_v, cross_hbm.at[pl.ds(base, _CB)])
        pltpu.sync_copy(lin_v, lin_hbm.at[pl.ds(base, _CB)])


_BLK = 2048


def _mlp_tc(cross_ref, lin_ref, w1_ref, b1_ref, w2_ref, b2_ref, w3_ref,
            c_ref, out_ref):
    x = cross_ref[...]
    h = jnp.dot(x, w1_ref[...], preferred_element_type=jnp.float32)
    h = jnp.maximum(h + b1_ref[...][None, :], 0.0)
    h = jnp.dot(h, w2_ref[...], preferred_element_type=jnp.float32)
    h = jnp.maximum(h + b2_ref[...][None, :], 0.0)
    o = jnp.dot(h, w3_ref[...], preferred_element_type=jnp.float32)[:, 0]
    out_ref[...] = jax.nn.sigmoid(o + lin_ref[...] + c_ref[0])


_mlp_call = pl.pallas_call(
    _mlp_tc,
    grid=(_B // _BLK,),
    in_specs=[
        pl.BlockSpec((_BLK, _E), lambda i: (i, 0)),
        pl.BlockSpec((_BLK,), lambda i: (i,)),
        pl.BlockSpec((_E, 128), lambda i: (0, 0)),
        pl.BlockSpec((128,), lambda i: (0,)),
        pl.BlockSpec((128, 64), lambda i: (0, 0)),
        pl.BlockSpec((64,), lambda i: (0,)),
        pl.BlockSpec((64, 1), lambda i: (0, 0)),
        pl.BlockSpec(memory_space=pltpu.SMEM),
    ],
    out_specs=pl.BlockSpec((_BLK,), lambda i: (i,)),
    out_shape=jax.ShapeDtypeStruct((_B,), jnp.float32),
)


def kernel(data, embed_table, fc_table, fc_bias, W1, b1, W2, b2, W3, b3):
    idx = data.astype(jnp.int32) + jnp.asarray(_OFF)[None, :]
    idx_t = idx.reshape(_NW, _NCHUNK, _CB, _F).transpose(0, 1, 3, 2).reshape(-1)
    tail_lin = lax.slice(embed_table, (_VMAIN, 0), (_V, _E)).reshape(-1)
    emb_lin = _detile_sc(embed_table.T, tail_lin).reshape(_V, _E)
    cross, lin = _fm_sc(idx_t, emb_lin, fc_table.reshape(-1))
    c = (fc_bias + b3).astype(jnp.float32)
    return _mlp_call(cross, lin, W1, b1, W2, b2, W3, c)


# speedup vs baseline: 1.3026x; 1.2932x over previous
import functools

import jax
import jax.numpy as jnp
import numpy as np
from jax import lax
from jax.experimental import pallas as pl
from jax.experimental.pallas import tpu as pltpu
from jax.experimental.pallas import tpu_sc as plsc

_FIELD_DIMS = [100000] * 26
_OFF = np.concatenate([[0], np.cumsum(_FIELD_DIMS)[:-1]]).astype(np.int32)
_B, _F, _E = 16384, 26, 16
_V = int(sum(_FIELD_DIMS))
_NC, _NS = 2, 16
_NW = _NC * _NS
_BPW = _B // _NW
_CB = 64
_NCHUNK = _BPW // _CB

_W = 1024
_VMAIN = (_V // _W) * _W
_NDCHUNK = _VMAIN // _W
_DPW = -(-_NDCHUNK // _NW)
_NBUF = 3
_DTRIP = -(-_DPW // _NBUF)
_TAIL = _V - _VMAIN


@functools.partial(
    pl.kernel,
    out_type=jax.ShapeDtypeStruct((_V * _E // 2,), jnp.int32),
    mesh=plsc.VectorSubcoreMesh(core_axis_name="c", subcore_axis_name="s"),
    compiler_params=pltpu.CompilerParams(
        use_tc_tiling_on_sc=True, needs_layout_passes=False),
    scratch_types=(
        [pltpu.VMEM((_E, _W), jnp.float32)] * _NBUF
        + [pltpu.VMEM((_W * _E // 2,), jnp.int32)] * _NBUF
        + [pltpu.SemaphoreType.DMA] * (2 * _NBUF)
    ),
)
def _detile_sc(emb_t, tail_lin, out_hbm, *bufs):
    wid = lax.axis_index("s") * _NC + lax.axis_index("c")
    chunks = bufs[:_NBUF]
    rows = bufs[_NBUF:2 * _NBUF]
    sems_i = bufs[2 * _NBUF:3 * _NBUF]
    sems_o = bufs[3 * _NBUF:4 * _NBUF]

    def cid_of(k):
        return k * _NW + wid

    def in_copy(k, b):
        return pltpu.make_async_copy(
            emb_t.at[:, pl.ds(cid_of(k) * _W, _W)], chunks[b], sems_i[b])

    def out_copy(k, b):
        off = pl.multiple_of(cid_of(k) * (_W * _E // 2), 8)
        return pltpu.make_async_copy(
            rows[b], out_hbm.at[pl.ds(off, _W * _E // 2)], sems_o[b])

    def transpose(b):
        def tbody(g, carry):
            j16 = g * 16
            pos = (j16 + lax.iota(jnp.int32, 16)) * (_E // 2)
            for e in range(0, _E, 2):
                lo = chunks[b][e, pl.ds(j16, 16)]
                hi = chunks[b][e + 1, pl.ds(j16, 16)]
                pk = plsc.bitcast(
                    plsc.pack(lo, hi, format=plsc.PackFormat.INTERLEAVED),
                    jnp.int32)
                plsc.store_scatter(rows[b], [pos + e // 2], pk)
            return carry

        lax.fori_loop(0, _W // 16, tbody, 0)

    for b in range(_NBUF):
        @pl.when(cid_of(b) < _NDCHUNK)
        def _():
            in_copy(b, b).start()

    def ring_body(m, carry):
        for b in range(_NBUF):
            k = _NBUF * m + b

            @pl.when(cid_of(k) < _NDCHUNK)
            def _():
                in_copy(k, b).wait()

                @pl.when(m > 0)
                def _():
                    out_copy(k - _NBUF, b).wait()

                transpose(b)
                out_copy(k, b).start()

                @pl.when(cid_of(k + _NBUF) < _NDCHUNK)
                def _():
                    in_copy(k + _NBUF, b).start()

        return carry

    lax.fori_loop(0, _DTRIP, ring_body, 0)

    for k in range(max(0, _NBUF * _DTRIP - 2 * _NBUF), _NBUF * _DTRIP):
        @pl.when((cid_of(k) < _NDCHUNK) & (cid_of(k + _NBUF) >= _NDCHUNK))
        def _():
            out_copy(k, k % _NBUF).wait()

    @pl.when(wid == _NW - 1)
    def _():
        pltpu.sync_copy(
            tail_lin, out_hbm.at[pl.ds(_VMAIN * _E // 2, _TAIL * _E // 2)])


@functools.partial(
    pl.kernel,
    out_type=(
        jax.ShapeDtypeStruct((_B * _E,), jnp.float32),
        jax.ShapeDtypeStruct((_B,), jnp.float32),
    ),
    mesh=plsc.VectorSubcoreMesh(core_axis_name="c", subcore_axis_name="s"),
    compiler_params=pltpu.CompilerParams(
        use_tc_tiling_on_sc=False, needs_layout_passes=False),
    scratch_types=(
        [pltpu.VMEM((_CB * _F,), jnp.int32)] * 2
        + [pltpu.VMEM((_CB * _F, _E // 2), jnp.int32)] * 2
        + [pltpu.VMEM((_CB * _F,), jnp.float32)] * 2
        + [
            pltpu.VMEM((_CB * _E,), jnp.float32),
            pltpu.VMEM((_CB,), jnp.float32),
        ]
        + [pltpu.SemaphoreType.DMA] * 4
    ),
)
def _fm_sc(idx_hbm, emb_hbm, fc_hbm, cross_hbm, lin_hbm,
           idx_v0, idx_v1, rows_v0, rows_v1, fc_v0, fc_v1, cross_v, lin_v,
           sem_e0, sem_e1, sem_f0, sem_f1):
    wid = lax.axis_index("s") * _NC + lax.axis_index("c")
    idxs = (idx_v0, idx_v1)
    rows = (rows_v0, rows_v1)
    fcs = (fc_v0, fc_v1)
    sems_e = (sem_e0, sem_e1)
    sems_f = (sem_f0, sem_f1)

    def stage(c, b):
        off = pl.multiple_of((wid * _NCHUNK + c) * _CB * _F, 8)
        pltpu.sync_copy(idx_hbm.at[pl.ds(off, _CB * _F)], idxs[b])
        pltpu.async_copy(emb_hbm.at[idxs[b]], rows[b], sems_e[b])
        pltpu.async_copy(fc_hbm.at[idxs[b]], fcs[b], sems_f[b])

    stage(0, 0)
    for c in range(_NCHUNK):
        b = c % 2
        base = wid * _BPW + c * _CB
        pltpu.make_async_copy(emb_hbm.at[idxs[b]], rows[b], sems_e[b]).wait()
        pltpu.make_async_copy(fc_hbm.at[idxs[b]], fcs[b], sems_f[b]).wait()
        if c + 1 < _NCHUNK:
            stage(c + 1, 1 - b)

        rows_v = rows[b]
        fc_v = fcs[b]
        lane = lax.iota(jnp.int32, 16)
        half = lane // 8
        i1 = lane % 8

        def fm_body(t, carry):
            bb = t * 2
            s_ev = jnp.zeros((16,), jnp.float32)
            s_od = jnp.zeros((16,), jnp.float32)
            sq_ev = jnp.zeros((16,), jnp.float32)
            sq_od = jnp.zeros((16,), jnp.float32)
            base = bb + half
            for f in range(_F):
                v = plsc.load_gather(rows_v, [base + f * _CB, i1])
                ev, od = plsc.unpack(plsc.bitcast(v, jnp.bfloat16),
                                     format=plsc.PackFormat.INTERLEAVED,
                                     preferred_element_type=jnp.float32)
                s_ev = s_ev + ev
                s_od = s_od + od
                sq_ev = sq_ev + ev * ev
                sq_od = sq_od + od * od
            pos = (bb + half) * _E + i1 * 2
            plsc.store_scatter(cross_v, [pos], 0.5 * (s_ev * s_ev - sq_ev))
            plsc.store_scatter(cross_v, [pos + 1], 0.5 * (s_od * s_od - sq_od))
            return carry

        lax.fori_loop(0, _CB // 2, fm_body, 0)

        def lin_body(g, carry):
            b0 = g * 16
            acc = fc_v[pl.ds(b0, 16)]
            for f in range(1, _F):
                acc = acc + fc_v[pl.ds(f * _CB + b0, 16)]
            lin_v[pl.ds(b0, 16)] = acc
            return carry

        lax.fori_loop(0, _CB // 16, lin_body, 0)

        pltpu.sync_copy(
            cross_v,
            cross_hbm.at[pl.ds(pl.multiple_of(base * _E, 8), _CB * _E)])
        pltpu.sync_copy(lin_v, lin_hbm.at[pl.ds(pl.multiple_of(base, 8), _CB)])


_BLK = 2048


def _mlp_tc(cross_ref, lin_ref, w1_ref, b1_ref, w2_ref, b2_ref, w3_ref,
            c_ref, out_ref):
    x = cross_ref[...]
    h = jnp.dot(x, w1_ref[...], preferred_element_type=jnp.float32)
    h = jnp.maximum(h + b1_ref[...][None, :], 0.0)
    h = jnp.dot(h, w2_ref[...], preferred_element_type=jnp.float32)
    h = jnp.maximum(h + b2_ref[...][None, :], 0.0)
    o = jnp.dot(h, w3_ref[...], preferred_element_type=jnp.float32)[:, 0]
    out_ref[...] = jax.nn.sigmoid(o + lin_ref[...] + c_ref[0])


_mlp_call = pl.pallas_call(
    _mlp_tc,
    grid=(_B // _BLK,),
    in_specs=[
        pl.BlockSpec((_BLK, _E), lambda i: (i, 0)),
        pl.BlockSpec((_BLK,), lambda i: (i,)),
        pl.BlockSpec((_E, 128), lambda i: (0, 0)),
        pl.BlockSpec((128,), lambda i: (0,)),
        pl.BlockSpec((128, 64), lambda i: (0, 0)),
        pl.BlockSpec((64,), lambda i: (0,)),
        pl.BlockSpec((64, 1), lambda i: (0, 0)),
        pl.BlockSpec(memory_space=pltpu.SMEM),
    ],
    out_specs=pl.BlockSpec((_BLK,), lambda i: (i,)),
    out_shape=jax.ShapeDtypeStruct((_B,), jnp.float32),
)


def kernel(data, embed_table, fc_table, fc_bias, W1, b1, W2, b2, W3, b3):
    idx = data.astype(jnp.int32) + jnp.asarray(_OFF)[None, :]
    idx_t = idx.reshape(_NW, _NCHUNK, _CB, _F).transpose(0, 1, 3, 2).reshape(-1)
    tail = lax.slice(embed_table, (_VMAIN, 0), (_V, _E)).astype(jnp.bfloat16)
    tail_pk = lax.bitcast_convert_type(
        tail.reshape(_TAIL * _E // 2, 2), jnp.int32)
    emb_pk = _detile_sc(embed_table.T, tail_pk).reshape(_V, _E // 2)
    cross_flat, lin = _fm_sc(idx_t, emb_pk, fc_table.reshape(-1))
    cross = cross_flat.reshape(_B, _E)
    c = (fc_bias + b3).astype(jnp.float32)
    return _mlp_call(cross, lin, W1, b1, W2, b2, W3, c)


# speedup vs baseline: 1.3280x; 1.0195x over previous
import functools

import jax
import jax.numpy as jnp
import numpy as np
from jax import lax
from jax.experimental import pallas as pl
from jax.experimental.pallas import tpu as pltpu
from jax.experimental.pallas import tpu_sc as plsc

_FIELD_DIMS = [100000] * 26
_OFF = np.concatenate([[0], np.cumsum(_FIELD_DIMS)[:-1]]).astype(np.int32)
_B, _F, _E = 16384, 26, 16
_V = int(sum(_FIELD_DIMS))
_NC, _NS = 2, 16
_NW = _NC * _NS
_BPW = _B // _NW
_CB = 128
_NCHUNK = _BPW // _CB

_W = 1024
_VMAIN = (_V // _W) * _W
_NDCHUNK = _VMAIN // _W
_DPW = -(-_NDCHUNK // _NW)
_NBUF = 3
_DTRIP = -(-_DPW // _NBUF)
_TAIL = _V - _VMAIN


@functools.partial(
    pl.kernel,
    out_type=jax.ShapeDtypeStruct((_V * _E // 2,), jnp.int32),
    mesh=plsc.VectorSubcoreMesh(core_axis_name="c", subcore_axis_name="s"),
    compiler_params=pltpu.CompilerParams(
        use_tc_tiling_on_sc=True, needs_layout_passes=False),
    scratch_types=(
        [pltpu.VMEM((_E, _W), jnp.float32)] * _NBUF
        + [pltpu.VMEM((_W * _E // 2,), jnp.int32)] * _NBUF
        + [pltpu.SemaphoreType.DMA] * (2 * _NBUF)
    ),
)
def _detile_sc(emb_t, tail_lin, out_hbm, *bufs):
    wid = lax.axis_index("s") * _NC + lax.axis_index("c")
    chunks = bufs[:_NBUF]
    rows = bufs[_NBUF:2 * _NBUF]
    sems_i = bufs[2 * _NBUF:3 * _NBUF]
    sems_o = bufs[3 * _NBUF:4 * _NBUF]

    def cid_of(k):
        return k * _NW + wid

    def in_copy(k, b):
        return pltpu.make_async_copy(
            emb_t.at[:, pl.ds(cid_of(k) * _W, _W)], chunks[b], sems_i[b])

    def out_copy(k, b):
        off = pl.multiple_of(cid_of(k) * (_W * _E // 2), 8)
        return pltpu.make_async_copy(
            rows[b], out_hbm.at[pl.ds(off, _W * _E // 2)], sems_o[b])

    def transpose(b):
        def tbody(g, carry):
            j16 = g * 16
            pos = (j16 + lax.iota(jnp.int32, 16)) * (_E // 2)
            for e in range(0, _E, 2):
                lo = chunks[b][e, pl.ds(j16, 16)]
                hi = chunks[b][e + 1, pl.ds(j16, 16)]
                pk = plsc.bitcast(
                    plsc.pack(lo, hi, format=plsc.PackFormat.INTERLEAVED),
                    jnp.int32)
                plsc.store_scatter(rows[b], [pos + e // 2], pk)
            return carry

        lax.fori_loop(0, _W // 16, tbody, 0)

    for b in range(_NBUF):
        @pl.when(cid_of(b) < _NDCHUNK)
        def _():
            in_copy(b, b).start()

    def ring_body(m, carry):
        for b in range(_NBUF):
            k = _NBUF * m + b

            @pl.when(cid_of(k) < _NDCHUNK)
            def _():
                in_copy(k, b).wait()

                @pl.when(m > 0)
                def _():
                    out_copy(k - _NBUF, b).wait()

                transpose(b)
                out_copy(k, b).start()

                @pl.when(cid_of(k + _NBUF) < _NDCHUNK)
                def _():
                    in_copy(k + _NBUF, b).start()

        return carry

    lax.fori_loop(0, _DTRIP, ring_body, 0)

    for k in range(max(0, _NBUF * _DTRIP - 2 * _NBUF), _NBUF * _DTRIP):
        @pl.when((cid_of(k) < _NDCHUNK) & (cid_of(k + _NBUF) >= _NDCHUNK))
        def _():
            out_copy(k, k % _NBUF).wait()

    @pl.when(wid == _NW - 1)
    def _():
        pltpu.sync_copy(
            tail_lin, out_hbm.at[pl.ds(_VMAIN * _E // 2, _TAIL * _E // 2)])


@functools.partial(
    pl.kernel,
    out_type=(
        jax.ShapeDtypeStruct((_B * _E,), jnp.float32),
        jax.ShapeDtypeStruct((_B,), jnp.float32),
    ),
    mesh=plsc.VectorSubcoreMesh(core_axis_name="c", subcore_axis_name="s"),
    compiler_params=pltpu.CompilerParams(
        use_tc_tiling_on_sc=False, needs_layout_passes=False),
    scratch_types=(
        [pltpu.VMEM((_CB * _F,), jnp.int32)] * 2
        + [pltpu.VMEM((_CB * _F, _E // 2), jnp.int32)] * 2
        + [pltpu.VMEM((_CB * _F,), jnp.float32)] * 2
        + [
            pltpu.VMEM((_CB * _E,), jnp.float32),
            pltpu.VMEM((_CB,), jnp.float32),
        ]
        + [pltpu.SemaphoreType.DMA] * 4
    ),
)
def _fm_sc(idx_hbm, emb_hbm, fc_hbm, cross_hbm, lin_hbm,
           idx_v0, idx_v1, rows_v0, rows_v1, fc_v0, fc_v1, cross_v, lin_v,
           sem_e0, sem_e1, sem_f0, sem_f1):
    wid = lax.axis_index("s") * _NC + lax.axis_index("c")
    idxs = (idx_v0, idx_v1)
    rows = (rows_v0, rows_v1)
    fcs = (fc_v0, fc_v1)
    sems_e = (sem_e0, sem_e1)
    sems_f = (sem_f0, sem_f1)

    def stage(c, b):
        off = pl.multiple_of((wid * _NCHUNK + c) * _CB * _F, 8)
        pltpu.sync_copy(idx_hbm.at[pl.ds(off, _CB * _F)], idxs[b])
        pltpu.async_copy(emb_hbm.at[idxs[b]], rows[b], sems_e[b])
        pltpu.async_copy(fc_hbm.at[idxs[b]], fcs[b], sems_f[b])

    stage(0, 0)
    for c in range(_NCHUNK):
        b = c % 2
        base = wid * _BPW + c * _CB
        pltpu.make_async_copy(emb_hbm.at[idxs[b]], rows[b], sems_e[b]).wait()
        pltpu.make_async_copy(fc_hbm.at[idxs[b]], fcs[b], sems_f[b]).wait()
        if c + 1 < _NCHUNK:
            stage(c + 1, 1 - b)

        rows_v = rows[b]
        fc_v = fcs[b]
        lane = lax.iota(jnp.int32, 16)
        half = lane // 8
        i1 = lane % 8

        def fm_body(t, carry):
            bb = t * 2
            s_ev = jnp.zeros((16,), jnp.float32)
            s_od = jnp.zeros((16,), jnp.float32)
            sq_ev = jnp.zeros((16,), jnp.float32)
            sq_od = jnp.zeros((16,), jnp.float32)
            base = bb + half
            for f in range(_F):
                v = plsc.load_gather(rows_v, [base + f * _CB, i1])
                ev, od = plsc.unpack(plsc.bitcast(v, jnp.bfloat16),
                                     format=plsc.PackFormat.INTERLEAVED,
                                     preferred_element_type=jnp.float32)
                s_ev = s_ev + ev
                s_od = s_od + od
                sq_ev = sq_ev + ev * ev
                sq_od = sq_od + od * od
            pos = (bb + half) * _E + i1 * 2
            plsc.store_scatter(cross_v, [pos], 0.5 * (s_ev * s_ev - sq_ev))
            plsc.store_scatter(cross_v, [pos + 1], 0.5 * (s_od * s_od - sq_od))
            return carry

        lax.fori_loop(0, _CB // 2, fm_body, 0)

        def lin_body(g, carry):
            b0 = g * 16
            acc = fc_v[pl.ds(b0, 16)]
            for f in range(1, _F):
                acc = acc + fc_v[pl.ds(f * _CB + b0, 16)]
            lin_v[pl.ds(b0, 16)] = acc
            return carry

        lax.fori_loop(0, _CB // 16, lin_body, 0)

        pltpu.sync_copy(
            cross_v,
            cross_hbm.at[pl.ds(pl.multiple_of(base * _E, 8), _CB * _E)])
        pltpu.sync_copy(lin_v, lin_hbm.at[pl.ds(pl.multiple_of(base, 8), _CB)])


_BLK = 2048


def _mlp_tc(cross_ref, lin_ref, w1_ref, b1_ref, w2_ref, b2_ref, w3_ref,
            c_ref, out_ref):
    x = cross_ref[...]
    h = jnp.dot(x, w1_ref[...], preferred_element_type=jnp.float32)
    h = jnp.maximum(h + b1_ref[...][None, :], 0.0)
    h = jnp.dot(h, w2_ref[...], preferred_element_type=jnp.float32)
    h = jnp.maximum(h + b2_ref[...][None, :], 0.0)
    o = jnp.dot(h, w3_ref[...], preferred_element_type=jnp.float32)[:, 0]
    out_ref[...] = jax.nn.sigmoid(o + lin_ref[...] + c_ref[0])


_mlp_call = pl.pallas_call(
    _mlp_tc,
    grid=(_B // _BLK,),
    in_specs=[
        pl.BlockSpec((_BLK, _E), lambda i: (i, 0)),
        pl.BlockSpec((_BLK,), lambda i: (i,)),
        pl.BlockSpec((_E, 128), lambda i: (0, 0)),
        pl.BlockSpec((128,), lambda i: (0,)),
        pl.BlockSpec((128, 64), lambda i: (0, 0)),
        pl.BlockSpec((64,), lambda i: (0,)),
        pl.BlockSpec((64, 1), lambda i: (0, 0)),
        pl.BlockSpec(memory_space=pltpu.SMEM),
    ],
    out_specs=pl.BlockSpec((_BLK,), lambda i: (i,)),
    out_shape=jax.ShapeDtypeStruct((_B,), jnp.float32),
)


def kernel(data, embed_table, fc_table, fc_bias, W1, b1, W2, b2, W3, b3):
    idx = data.astype(jnp.int32) + jnp.asarray(_OFF)[None, :]
    idx_t = idx.reshape(_NW, _NCHUNK, _CB, _F).transpose(0, 1, 3, 2).reshape(-1)
    tail = lax.slice(embed_table, (_VMAIN, 0), (_V, _E)).astype(jnp.bfloat16)
    tail_pk = lax.bitcast_convert_type(
        tail.reshape(_TAIL * _E // 2, 2), jnp.int32)
    emb_pk = _detile_sc(embed_table.T, tail_pk).reshape(_V, _E // 2)
    cross_flat, lin = _fm_sc(idx_t, emb_pk, fc_table.reshape(-1))
    cross = cross_flat.reshape(_B, _E)
    c = (fc_bias + b3).astype(jnp.float32)
    return _mlp_call(cross, lin, W1, b1, W2, b2, W3, c)
